# baseline (speedup 1.0000x reference)
import functools

import jax
import jax.numpy as jnp
from jax import lax
from jax.experimental import pallas as pl
from jax.experimental.pallas import tpu as pltpu
from jax.experimental.pallas import tpu_sc as plsc

N = 10000
E = 320000
D = 128
H = 128
C = 2
B = 64

AW = 144
NC = 2
NS = 16
NW = NC * NS
EPW = E // NW
CHUNK = 40
NCHUNK = EPW // CHUNK
NP = 10240
RPT = NP // NS

RB = 1000
NRB = N // RB
CP = 8
CW = 8


def _sc_aggregate(x, src_r, dst_r, const8):
    mesh = plsc.VectorSubcoreMesh(core_axis_name="c", subcore_axis_name="s")

    @functools.partial(
        pl.kernel,
        mesh=mesh,
        compiler_params=pltpu.CompilerParams(use_tc_tiling_on_sc=False),
        out_type=[jax.ShapeDtypeStruct((NC, NP, D), jnp.float32),
                  jax.ShapeDtypeStruct((NC, NP, CW), jnp.float32)],
        scratch_types=[
            pltpu.VMEM((NCHUNK, CHUNK), jnp.int32),
            pltpu.VMEM((NCHUNK, CHUNK), jnp.int32),
            pltpu.VMEM((CHUNK, D), jnp.float32),
            pltpu.VMEM((CHUNK, D), jnp.float32),
            pltpu.VMEM((CHUNK, D), jnp.float32),
            pltpu.VMEM((CHUNK, D), jnp.float32),
            pltpu.VMEM((2 * CHUNK, CW), jnp.float32),
            pltpu.VMEM_SHARED((NP, D), jnp.float32),
            pltpu.VMEM_SHARED((NP, CW), jnp.float32),
            pltpu.SemaphoreType.DMA,
            pltpu.SemaphoreType.DMA,
            pltpu.SemaphoreType.DMA,
            pltpu.SemaphoreType.DMA,
            pltpu.SemaphoreType.DMA,
            pltpu.SemaphoreType.DMA,
            pltpu.SemaphoreType.DMA,
            pltpu.SemaphoreType.DMA,
            pltpu.SemaphoreType.DMA,
        ],
    )
    def k(x_hbm, src_hbm, dst_hbm, c8_hbm, out_hbm, cnt_hbm, src_v, dst_v,
          buf0, buf1, buf2, buf3, obuf, acc_sh, cnt_sh,
          gs0, gs1, gs2, gs3, ss0, ss1, ss2, ss3, csem):
        c = lax.axis_index("c")
        s = lax.axis_index("s")
        wid = c * NS + s

        zero16 = jnp.zeros((16,), jnp.float32)

        pltpu.sync_copy(c8_hbm, obuf)
        zrows = obuf.at[pl.ds(0, CHUNK)]
        ones = obuf.at[pl.ds(CHUNK, CHUNK)]

        def zb(kk, carry):
            buf0[kk // (D // 16), pl.ds((kk % (D // 16)) * 16, 16)] = zero16
            return carry

        lax.fori_loop(0, CHUNK * (D // 16), zb, 0)

        def zacc(kk, carry):
            pltpu.sync_copy(buf0, acc_sh.at[pl.ds(s * RPT + kk * CHUNK, CHUNK)])
            pltpu.sync_copy(zrows, cnt_sh.at[pl.ds(s * RPT + kk * CHUNK, CHUNK)])
            return carry

        lax.fori_loop(0, RPT // CHUNK, zacc, 0)

        pltpu.sync_copy(src_hbm.at[wid], src_v)
        pltpu.sync_copy(dst_hbm.at[wid], dst_v)
        plsc.subcore_barrier()

        bufs = (buf0, buf1, buf2, buf3)
        gsems = (gs0, gs1, gs2, gs3)
        ssems = (ss0, ss1, ss2, ss3)

        def issue(j, b_, sem_):
            pltpu.async_copy(x_hbm.at[src_v.at[j]], b_, sem_)

        def wait_g(b_, sem_):
            pltpu.make_async_copy(x_hbm.at[src_v.at[0]], b_, sem_).wait()

        def scat(j, b_, sem_):
            pltpu.async_copy(b_, acc_sh.at[dst_v.at[j]], sem_, add=True)

        def wait_s(b_, sem_):
            pltpu.make_async_copy(b_, acc_sh.at[dst_v.at[0]], sem_).wait()

        def cnt_issue(j):
            pltpu.async_copy(ones, cnt_sh.at[dst_v.at[j]], csem, add=True)

        def cnt_wait(kk=0, carry=0):
            pltpu.make_async_copy(ones, cnt_sh.at[dst_v.at[0]], csem).wait()
            return carry

        issue(0, buf0, gs0)
        issue(1, buf1, gs1)
        issue(2, buf2, gs2)
        issue(3, buf3, gs3)

        def turn(j, k, with_issue):
            wait_g(bufs[k], gsems[k])
            scat(j, bufs[k], ssems[k])
            cnt_issue(j)
            if with_issue:
                k2 = (k + 2) % 4
                wait_s(bufs[k2], ssems[k2])
                issue(j + 2, bufs[k2], gsems[k2])

        turn(0, 0, False)
        turn(1, 1, False)
        turn(2, 2, True)
        turn(3, 3, True)

        def body(i, carry):
            j = 4 * i
            turn(j, 0, True)
            turn(j + 1, 1, True)
            turn(j + 2, 2, True)
            turn(j + 3, 3, True)
            return carry

        lax.fori_loop(1, (NCHUNK - 2) // 4, body, 0)
        turn(NCHUNK - 2, 0, False)
        turn(NCHUNK - 1, 1, False)
        wait_s(bufs[2], ssems[2])
        wait_s(bufs[3], ssems[3])
        wait_s(bufs[0], ssems[0])
        wait_s(bufs[1], ssems[1])
        lax.fori_loop(0, NCHUNK, cnt_wait, 0)
        plsc.subcore_barrier()

        pltpu.sync_copy(acc_sh.at[pl.ds(s * RPT, RPT)],
                        out_hbm.at[c, pl.ds(s * RPT, RPT)])
        pltpu.sync_copy(cnt_sh.at[pl.ds(s * RPT, RPT)],
                        cnt_hbm.at[c, pl.ds(s * RPT, RPT)])

    return k(x, src_r, dst_r, const8)


def _tc_body(bounds_ref, pf_ref, pc_ref, x_ref, batch_ref, wlt_ref, wrt_ref,
             bl_ref, w2t_ref, b2_ref, out_ref, acc_ref):
    i = pl.program_id(0)

    @pl.when(i == 0)
    def _():
        acc_ref[...] = jnp.zeros_like(acc_ref)

    ssum = pf_ref[0] + pf_ref[1]
    pc = pc_ref[0] + pc_ref[1]
    cnt = pc[:, :1]
    mean = ssum / jnp.maximum(cnt, 1.0)
    z = jnp.dot(mean, wlt_ref[...], preferred_element_type=jnp.float32)
    z = z + jnp.dot(x_ref[...], wrt_ref[...], preferred_element_type=jnp.float32)
    z = z + bl_ref[...]

    bcol = batch_ref[0]
    gmin = bounds_ref[0, i]
    gmax = bounds_ref[1, i]
    for off in range(B):
        g = gmin + off

        @pl.when(g <= gmax)
        def _(g=g):
            zm = jnp.where(bcol == g, z, 0.0)
            contrib = jnp.max(zm, axis=0, keepdims=True)
            cur = acc_ref[pl.ds(g, 1), :]
            acc_ref[pl.ds(g, 1), :] = jnp.maximum(cur, contrib)

    @pl.when(i == NRB - 1)
    def _():
        pooled = acc_ref[...]
        logits = jnp.dot(pooled, w2t_ref[...],
                         preferred_element_type=jnp.float32) + b2_ref[...]
        col = lax.broadcasted_iota(jnp.int32, (B, CP), 1)
        logits = jnp.where(col < C, logits, -jnp.inf)
        mx = jnp.max(logits, axis=-1, keepdims=True)
        sh = logits - mx
        lse = jnp.log(jnp.sum(jnp.exp(sh), axis=-1, keepdims=True))
        out_ref[...] = (sh - lse)[:, :C]


def _tc_head(bounds, pfeat, pcnt, x, batch3, wlt, wrt, bl, w2t, b2p):
    grid_spec = pltpu.PrefetchScalarGridSpec(
        num_scalar_prefetch=1,
        grid=(NRB,),
        in_specs=[
            pl.BlockSpec((NC, RB, D), lambda i, b_: (0, i, 0)),
            pl.BlockSpec((NC, RB, CW), lambda i, b_: (0, i, 0)),
            pl.BlockSpec((RB, D), lambda i, b_: (i, 0)),
            pl.BlockSpec((1, RB, 1), lambda i, b_: (i, 0, 0)),
            pl.BlockSpec((D, H), lambda i, b_: (0, 0)),
            pl.BlockSpec((D, H), lambda i, b_: (0, 0)),
            pl.BlockSpec((1, H), lambda i, b_: (0, 0)),
            pl.BlockSpec((H, CP), lambda i, b_: (0, 0)),
            pl.BlockSpec((1, CP), lambda i, b_: (0, 0)),
        ],
        out_specs=pl.BlockSpec((B, C), lambda i, b_: (0, 0)),
        scratch_shapes=[pltpu.VMEM((B, H), jnp.float32)],
    )
    return pl.pallas_call(
        _tc_body,
        grid_spec=grid_spec,
        out_shape=jax.ShapeDtypeStruct((B, C), jnp.float32),
    )(bounds, pfeat, pcnt, x, batch3, wlt, wrt, bl, w2t, b2p)


def kernel(x, edge_index, batch, W_l, b_l, W_r, W2, b2):
    src_r = edge_index[0].reshape(NW, NCHUNK, CHUNK)
    dst_r = edge_index[1].reshape(NW, NCHUNK, CHUNK)

    const8 = jnp.concatenate(
        [jnp.zeros((CHUNK, CW), jnp.float32),
         jnp.zeros((CHUNK, CW), jnp.float32).at[:, 0].set(1.0)], axis=0)
    pfeat, pcnt = _sc_aggregate(x, src_r, dst_r, const8)

    batch2 = batch.reshape(NRB, RB)
    bounds = jnp.stack([batch2[:, 0], batch2[:, -1]])
    batch3 = batch.reshape(NRB, RB, 1)
    wlt = W_l.T
    wrt = W_r.T
    bl = b_l.reshape(1, H)
    w2t = jnp.zeros((H, CP), jnp.float32).at[:, :C].set(W2.T)
    b2p = jnp.zeros((1, CP), jnp.float32).at[0, :C].set(b2)

    return _tc_head(bounds, pfeat, pcnt, x, batch3, wlt, wrt, bl, w2t, b2p)


# speedup vs baseline: 1.1081x; 1.1081x over previous
import functools

import jax
import jax.numpy as jnp
from jax import lax
from jax.experimental import pallas as pl
from jax.experimental.pallas import tpu as pltpu
from jax.experimental.pallas import tpu_sc as plsc

N = 10000
E = 320000
D = 128
H = 128
C = 2
B = 64

AW = 144
NC = 2
NS = 16
NW = NC * NS
EPW = E // NW
CHUNK = 80
NCHUNK = EPW // CHUNK
NP = 10240
RPT = NP // NS

RB = 1000
NRB = N // RB
CP = 8
CW = 8


def _sc_aggregate(x, src_r, dst_r, const8):
    mesh = plsc.VectorSubcoreMesh(core_axis_name="c", subcore_axis_name="s")

    @functools.partial(
        pl.kernel,
        mesh=mesh,
        compiler_params=pltpu.CompilerParams(use_tc_tiling_on_sc=False),
        out_type=[jax.ShapeDtypeStruct((NC, NP, D), jnp.float32),
                  jax.ShapeDtypeStruct((NC, NP, CW), jnp.float32)],
        scratch_types=[
            pltpu.VMEM((NCHUNK, CHUNK), jnp.int32),
            pltpu.VMEM((NCHUNK, CHUNK), jnp.int32),
            pltpu.VMEM((CHUNK, D), jnp.float32),
            pltpu.VMEM((CHUNK, D), jnp.float32),
            pltpu.VMEM((2 * CHUNK, CW), jnp.float32),
            pltpu.VMEM_SHARED((NP, D), jnp.float32),
            pltpu.VMEM_SHARED((NP, CW), jnp.float32),
            pltpu.SemaphoreType.DMA,
            pltpu.SemaphoreType.DMA,
            pltpu.SemaphoreType.DMA,
        ],
    )
    def k(x_hbm, src_hbm, dst_hbm, c8_hbm, out_hbm, cnt_hbm, src_v, dst_v,
          buf0, buf1, obuf, acc_sh, cnt_sh, gs0, gs1, csem):
        c = lax.axis_index("c")
        s = lax.axis_index("s")
        wid = c * NS + s

        zero16 = jnp.zeros((16,), jnp.float32)

        pltpu.sync_copy(c8_hbm, obuf)
        zrows = obuf.at[pl.ds(0, CHUNK)]
        ones = obuf.at[pl.ds(CHUNK, CHUNK)]

        def zb(kk, carry):
            buf0[kk // (D // 16), pl.ds((kk % (D // 16)) * 16, 16)] = zero16
            return carry

        lax.fori_loop(0, CHUNK * (D // 16), zb, 0)

        def zacc(kk, carry):
            pltpu.sync_copy(buf0, acc_sh.at[pl.ds(s * RPT + kk * CHUNK, CHUNK)])
            pltpu.sync_copy(zrows, cnt_sh.at[pl.ds(s * RPT + kk * CHUNK, CHUNK)])
            return carry

        lax.fori_loop(0, RPT // CHUNK, zacc, 0)

        pltpu.sync_copy(src_hbm.at[wid], src_v)
        pltpu.sync_copy(dst_hbm.at[wid], dst_v)
        plsc.subcore_barrier()

        def issue(j, b_, sem_):
            pltpu.async_copy(x_hbm.at[src_v.at[j]], b_, sem_)

        def wait_g(b_, sem_):
            pltpu.make_async_copy(x_hbm.at[src_v.at[0]], b_, sem_).wait()

        def scat(j, b_):
            pltpu.sync_copy(b_, acc_sh.at[dst_v.at[j]], add=True)

        def cnt_issue(j):
            pltpu.async_copy(ones, cnt_sh.at[dst_v.at[j]], csem, add=True)

        def cnt_wait(kk=0, carry=0):
            pltpu.make_async_copy(ones, cnt_sh.at[dst_v.at[0]], csem).wait()
            return carry

        issue(0, buf0, gs0)
        issue(1, buf1, gs1)

        def body(i, carry):
            j = 2 * i
            wait_g(buf0, gs0)
            scat(j, buf0)
            issue(j + 2, buf0, gs0)
            cnt_issue(j)
            wait_g(buf1, gs1)
            scat(j + 1, buf1)
            issue(j + 3, buf1, gs1)
            cnt_issue(j + 1)
            return carry

        lax.fori_loop(0, (NCHUNK - 3) // 2, body, 0)
        wait_g(buf0, gs0)
        scat(NCHUNK - 3, buf0)
        issue(NCHUNK - 1, buf0, gs0)
        cnt_issue(NCHUNK - 3)
        wait_g(buf1, gs1)
        scat(NCHUNK - 2, buf1)
        cnt_issue(NCHUNK - 2)
        wait_g(buf0, gs0)
        scat(NCHUNK - 1, buf0)
        cnt_issue(NCHUNK - 1)
        lax.fori_loop(0, NCHUNK, cnt_wait, 0)
        plsc.subcore_barrier()

        pltpu.sync_copy(acc_sh.at[pl.ds(s * RPT, RPT)],
                        out_hbm.at[c, pl.ds(s * RPT, RPT)])
        pltpu.sync_copy(cnt_sh.at[pl.ds(s * RPT, RPT)],
                        cnt_hbm.at[c, pl.ds(s * RPT, RPT)])

    return k(x, src_r, dst_r, const8)


def _tc_body(bounds_ref, pf_ref, pc_ref, x_ref, batch_ref, wlt_ref, wrt_ref,
             bl_ref, w2t_ref, b2_ref, out_ref, acc_ref):
    i = pl.program_id(0)

    @pl.when(i == 0)
    def _():
        acc_ref[...] = jnp.zeros_like(acc_ref)

    ssum = pf_ref[0] + pf_ref[1]
    pc = pc_ref[0] + pc_ref[1]
    cnt = pc[:, :1]
    mean = ssum / jnp.maximum(cnt, 1.0)
    z = jnp.dot(mean, wlt_ref[...], preferred_element_type=jnp.float32)
    z = z + jnp.dot(x_ref[...], wrt_ref[...], preferred_element_type=jnp.float32)
    z = z + bl_ref[...]

    bcol = batch_ref[0]
    gmin = bounds_ref[0, i]
    gmax = bounds_ref[1, i]
    for off in range(B):
        g = gmin + off

        @pl.when(g <= gmax)
        def _(g=g):
            zm = jnp.where(bcol == g, z, 0.0)
            contrib = jnp.max(zm, axis=0, keepdims=True)
            cur = acc_ref[pl.ds(g, 1), :]
            acc_ref[pl.ds(g, 1), :] = jnp.maximum(cur, contrib)

    @pl.when(i == NRB - 1)
    def _():
        pooled = acc_ref[...]
        logits = jnp.dot(pooled, w2t_ref[...],
                         preferred_element_type=jnp.float32) + b2_ref[...]
        col = lax.broadcasted_iota(jnp.int32, (B, CP), 1)
        logits = jnp.where(col < C, logits, -jnp.inf)
        mx = jnp.max(logits, axis=-1, keepdims=True)
        sh = logits - mx
        lse = jnp.log(jnp.sum(jnp.exp(sh), axis=-1, keepdims=True))
        out_ref[...] = (sh - lse)[:, :C]


def _tc_head(bounds, pfeat, pcnt, x, batch3, wlt, wrt, bl, w2t, b2p):
    grid_spec = pltpu.PrefetchScalarGridSpec(
        num_scalar_prefetch=1,
        grid=(NRB,),
        in_specs=[
            pl.BlockSpec((NC, RB, D), lambda i, b_: (0, i, 0)),
            pl.BlockSpec((NC, RB, CW), lambda i, b_: (0, i, 0)),
            pl.BlockSpec((RB, D), lambda i, b_: (i, 0)),
            pl.BlockSpec((1, RB, 1), lambda i, b_: (i, 0, 0)),
            pl.BlockSpec((D, H), lambda i, b_: (0, 0)),
            pl.BlockSpec((D, H), lambda i, b_: (0, 0)),
            pl.BlockSpec((1, H), lambda i, b_: (0, 0)),
            pl.BlockSpec((H, CP), lambda i, b_: (0, 0)),
            pl.BlockSpec((1, CP), lambda i, b_: (0, 0)),
        ],
        out_specs=pl.BlockSpec((B, C), lambda i, b_: (0, 0)),
        scratch_shapes=[pltpu.VMEM((B, H), jnp.float32)],
    )
    return pl.pallas_call(
        _tc_body,
        grid_spec=grid_spec,
        out_shape=jax.ShapeDtypeStruct((B, C), jnp.float32),
    )(bounds, pfeat, pcnt, x, batch3, wlt, wrt, bl, w2t, b2p)


def kernel(x, edge_index, batch, W_l, b_l, W_r, W2, b2):
    src_r = edge_index[0].reshape(NW, NCHUNK, CHUNK)
    dst_r = edge_index[1].reshape(NW, NCHUNK, CHUNK)

    const8 = jnp.concatenate(
        [jnp.zeros((CHUNK, CW), jnp.float32),
         jnp.zeros((CHUNK, CW), jnp.float32).at[:, 0].set(1.0)], axis=0)
    pfeat, pcnt = _sc_aggregate(x, src_r, dst_r, const8)

    batch2 = batch.reshape(NRB, RB)
    bounds = jnp.stack([batch2[:, 0], batch2[:, -1]])
    batch3 = batch.reshape(NRB, RB, 1)
    wlt = W_l.T
    wrt = W_r.T
    bl = b_l.reshape(1, H)
    w2t = jnp.zeros((H, CP), jnp.float32).at[:, :C].set(W2.T)
    b2p = jnp.zeros((1, CP), jnp.float32).at[0, :C].set(b2)

    return _tc_head(bounds, pfeat, pcnt, x, batch3, wlt, wrt, bl, w2t, b2p)


# speedup vs baseline: 1.2303x; 1.1103x over previous
import functools

import jax
import jax.numpy as jnp
from jax import lax
from jax.experimental import pallas as pl
from jax.experimental.pallas import tpu as pltpu
from jax.experimental.pallas import tpu_sc as plsc

N = 10000
E = 320000
D = 128
H = 128
C = 2
B = 64

AW = 144
NC = 2
NS = 16
NW = NC * NS
EPW = E // NW
CHUNK = 40
NCHUNK = EPW // CHUNK
NP = 10240
RPT = NP // NS

RB = 1000
NRB = N // RB
CP = 8
CW = 8


def _sc_aggregate(x, src_r, dst_r, const8):
    mesh = plsc.VectorSubcoreMesh(core_axis_name="c", subcore_axis_name="s")

    @functools.partial(
        pl.kernel,
        mesh=mesh,
        compiler_params=pltpu.CompilerParams(use_tc_tiling_on_sc=False),
        out_type=[jax.ShapeDtypeStruct((NC, NP, D), jnp.float32),
                  jax.ShapeDtypeStruct((NC, NP, CW), jnp.float32)],
        scratch_types=[
            pltpu.VMEM((NCHUNK, CHUNK), jnp.int32),
            pltpu.VMEM((NCHUNK, CHUNK), jnp.int32),
            pltpu.VMEM((CHUNK, D), jnp.float32),
            pltpu.VMEM((CHUNK, D), jnp.float32),
            pltpu.VMEM((CHUNK, D), jnp.float32),
            pltpu.VMEM((CHUNK, D), jnp.float32),
            pltpu.VMEM((2 * CHUNK, CW), jnp.float32),
            pltpu.VMEM_SHARED((NP, D), jnp.float32),
            pltpu.VMEM_SHARED((NP, CW), jnp.float32),
            pltpu.SemaphoreType.DMA,
            pltpu.SemaphoreType.DMA,
            pltpu.SemaphoreType.DMA,
            pltpu.SemaphoreType.DMA,
            pltpu.SemaphoreType.DMA,
        ],
    )
    def k(x_hbm, src_hbm, dst_hbm, c8_hbm, out_hbm, cnt_hbm, src_v, dst_v,
          buf0, buf1, buf2, buf3, obuf, acc_sh, cnt_sh, gs0, gs1, gs2, gs3, csem):
        c = lax.axis_index("c")
        s = lax.axis_index("s")
        wid = c * NS + s

        zero16 = jnp.zeros((16,), jnp.float32)

        pltpu.sync_copy(c8_hbm, obuf)
        zrows = obuf.at[pl.ds(0, CHUNK)]
        ones = obuf.at[pl.ds(CHUNK, CHUNK)]

        def zb(kk, carry):
            buf0[kk // (D // 16), pl.ds((kk % (D // 16)) * 16, 16)] = zero16
            return carry

        lax.fori_loop(0, CHUNK * (D // 16), zb, 0)

        def zacc(kk, carry):
            pltpu.sync_copy(buf0, acc_sh.at[pl.ds(s * RPT + kk * CHUNK, CHUNK)])
            pltpu.sync_copy(zrows, cnt_sh.at[pl.ds(s * RPT + kk * CHUNK, CHUNK)])
            return carry

        lax.fori_loop(0, RPT // CHUNK, zacc, 0)

        pltpu.sync_copy(src_hbm.at[wid], src_v)
        pltpu.sync_copy(dst_hbm.at[wid], dst_v)
        plsc.subcore_barrier()

        def issue(j, b_, sem_):
            pltpu.async_copy(x_hbm.at[src_v.at[j]], b_, sem_)

        def wait_g(b_, sem_):
            pltpu.make_async_copy(x_hbm.at[src_v.at[0]], b_, sem_).wait()

        def scat(j, b_):
            pltpu.sync_copy(b_, acc_sh.at[dst_v.at[j]], add=True)

        def cnt_issue(j):
            pltpu.async_copy(ones, cnt_sh.at[dst_v.at[j]], csem, add=True)

        def cnt_wait(kk=0, carry=0):
            pltpu.make_async_copy(ones, cnt_sh.at[dst_v.at[0]], csem).wait()
            return carry

        bufs = (buf0, buf1, buf2, buf3)
        gsems = (gs0, gs1, gs2, gs3)
        issue(0, buf0, gs0)
        issue(1, buf1, gs1)
        issue(2, buf2, gs2)
        issue(3, buf3, gs3)

        def turn(j, k, nxt):
            wait_g(bufs[k], gsems[k])
            scat(j, bufs[k])
            if nxt:
                issue(j + 4, bufs[k], gsems[k])
            cnt_issue(j)

        def body(i, carry):
            j = 4 * i
            turn(j, 0, True)
            turn(j + 1, 1, True)
            turn(j + 2, 2, True)
            turn(j + 3, 3, True)
            return carry

        lax.fori_loop(0, (NCHUNK - 6) // 4, body, 0)
        turn(NCHUNK - 6, 0, True)
        turn(NCHUNK - 5, 1, True)
        turn(NCHUNK - 4, 2, False)
        turn(NCHUNK - 3, 3, False)
        turn(NCHUNK - 2, 0, False)
        turn(NCHUNK - 1, 1, False)
        lax.fori_loop(0, NCHUNK, cnt_wait, 0)
        plsc.subcore_barrier()

        pltpu.sync_copy(acc_sh.at[pl.ds(s * RPT, RPT)],
                        out_hbm.at[c, pl.ds(s * RPT, RPT)])
        pltpu.sync_copy(cnt_sh.at[pl.ds(s * RPT, RPT)],
                        cnt_hbm.at[c, pl.ds(s * RPT, RPT)])

    return k(x, src_r, dst_r, const8)


def _tc_body(bounds_ref, pf_ref, pc_ref, x_ref, batch_ref, wlt_ref, wrt_ref,
             bl_ref, w2t_ref, b2_ref, out_ref, acc_ref):
    i = pl.program_id(0)

    @pl.when(i == 0)
    def _():
        acc_ref[...] = jnp.zeros_like(acc_ref)

    ssum = pf_ref[0] + pf_ref[1]
    pc = pc_ref[0] + pc_ref[1]
    cnt = pc[:, :1]
    mean = ssum / jnp.maximum(cnt, 1.0)
    z = jnp.dot(mean, wlt_ref[...], preferred_element_type=jnp.float32)
    z = z + jnp.dot(x_ref[...], wrt_ref[...], preferred_element_type=jnp.float32)
    z = z + bl_ref[...]

    bcol = batch_ref[0]
    gmin = bounds_ref[0, i]
    gmax = bounds_ref[1, i]
    for off in range(B):
        g = gmin + off

        @pl.when(g <= gmax)
        def _(g=g):
            zm = jnp.where(bcol == g, z, 0.0)
            contrib = jnp.max(zm, axis=0, keepdims=True)
            cur = acc_ref[pl.ds(g, 1), :]
            acc_ref[pl.ds(g, 1), :] = jnp.maximum(cur, contrib)

    @pl.when(i == NRB - 1)
    def _():
        pooled = acc_ref[...]
        logits = jnp.dot(pooled, w2t_ref[...],
                         preferred_element_type=jnp.float32) + b2_ref[...]
        col = lax.broadcasted_iota(jnp.int32, (B, CP), 1)
        logits = jnp.where(col < C, logits, -jnp.inf)
        mx = jnp.max(logits, axis=-1, keepdims=True)
        sh = logits - mx
        lse = jnp.log(jnp.sum(jnp.exp(sh), axis=-1, keepdims=True))
        out_ref[...] = (sh - lse)[:, :C]


def _tc_head(bounds, pfeat, pcnt, x, batch3, wlt, wrt, bl, w2t, b2p):
    grid_spec = pltpu.PrefetchScalarGridSpec(
        num_scalar_prefetch=1,
        grid=(NRB,),
        in_specs=[
            pl.BlockSpec((NC, RB, D), lambda i, b_: (0, i, 0)),
            pl.BlockSpec((NC, RB, CW), lambda i, b_: (0, i, 0)),
            pl.BlockSpec((RB, D), lambda i, b_: (i, 0)),
            pl.BlockSpec((1, RB, 1), lambda i, b_: (i, 0, 0)),
            pl.BlockSpec((D, H), lambda i, b_: (0, 0)),
            pl.BlockSpec((D, H), lambda i, b_: (0, 0)),
            pl.BlockSpec((1, H), lambda i, b_: (0, 0)),
            pl.BlockSpec((H, CP), lambda i, b_: (0, 0)),
            pl.BlockSpec((1, CP), lambda i, b_: (0, 0)),
        ],
        out_specs=pl.BlockSpec((B, C), lambda i, b_: (0, 0)),
        scratch_shapes=[pltpu.VMEM((B, H), jnp.float32)],
    )
    return pl.pallas_call(
        _tc_body,
        grid_spec=grid_spec,
        out_shape=jax.ShapeDtypeStruct((B, C), jnp.float32),
    )(bounds, pfeat, pcnt, x, batch3, wlt, wrt, bl, w2t, b2p)


def kernel(x, edge_index, batch, W_l, b_l, W_r, W2, b2):
    src_r = edge_index[0].reshape(NW, NCHUNK, CHUNK)
    dst_r = edge_index[1].reshape(NW, NCHUNK, CHUNK)

    const8 = jnp.concatenate(
        [jnp.zeros((CHUNK, CW), jnp.float32),
         jnp.zeros((CHUNK, CW), jnp.float32).at[:, 0].set(1.0)], axis=0)
    pfeat, pcnt = _sc_aggregate(x, src_r, dst_r, const8)

    batch2 = batch.reshape(NRB, RB)
    bounds = jnp.stack([batch2[:, 0], batch2[:, -1]])
    batch3 = batch.reshape(NRB, RB, 1)
    wlt = W_l.T
    wrt = W_r.T
    bl = b_l.reshape(1, H)
    w2t = jnp.zeros((H, CP), jnp.float32).at[:, :C].set(W2.T)
    b2p = jnp.zeros((1, CP), jnp.float32).at[0, :C].set(b2)

    return _tc_head(bounds, pfeat, pcnt, x, batch3, wlt, wrt, bl, w2t, b2p)
